# baseline (speedup 1.0000x reference)
import functools

import jax
import jax.numpy as jnp
from jax import lax
from jax.experimental import pallas as pl
from jax.experimental.pallas import tpu as pltpu
from jax.experimental.pallas import tpu_sc as plsc

N = 10000
E = 320000
D = 128

NC = 2
NS = 16
NW = NC * NS

NPAD = 10240
EMB_PER_W = NPAD // NW
EMB_CH = 80
EMB_NCH = EMB_PER_W // EMB_CH

EPT = E // NW
CH = 40
NCHUNK = EPT // CH
WB = 624
REM = N - NS * WB

_mesh = plsc.VectorSubcoreMesh(core_axis_name="c", subcore_axis_name="s")


@functools.partial(
    pl.kernel,
    out_type=jax.ShapeDtypeStruct((NPAD, D), jnp.float32),
    mesh=_mesh,
    scratch_types=[
        pltpu.VMEM((EMB_CH,), jnp.int32),
        pltpu.VMEM((EMB_CH, D), jnp.float32),
        pltpu.SemaphoreType.DMA,
    ],
)
def _embed_sc(emb_hbm, z_hbm, out_hbm, zi_v, row_v, sem):
    c = lax.axis_index("c")
    s = lax.axis_index("s")
    w = s * NC + c
    base = w * EMB_PER_W
    for k in range(EMB_NCH):
        pltpu.sync_copy(z_hbm.at[pl.ds(base + k * EMB_CH, EMB_CH)], zi_v)
        pltpu.async_copy(emb_hbm.at[zi_v], row_v, sem).wait()
        pltpu.sync_copy(row_v, out_hbm.at[pl.ds(base + k * EMB_CH, EMB_CH)])


@functools.partial(
    pl.kernel,
    out_type=jax.ShapeDtypeStruct((NC, N, D), jnp.float32),
    mesh=_mesh,
    scratch_types=[
        pltpu.VMEM((8, CH), jnp.int32),
        pltpu.VMEM((8, CH), jnp.int32),
        pltpu.VMEM((CH, D), jnp.float32),
        pltpu.VMEM((CH, D), jnp.float32),
        pltpu.VMEM((CH // 2, D), jnp.int32),
        pltpu.VMEM((CH // 2, D), jnp.int32),
        pltpu.VMEM((CH, D), jnp.float32),
        pltpu.VMEM((CH, D), jnp.float32),
        pltpu.VMEM_SHARED((N, D), jnp.float32),
        pltpu.SemaphoreType.DMA,
        pltpu.SemaphoreType.DMA,
        pltpu.SemaphoreType.DMA,
        pltpu.SemaphoreType.DMA,
        pltpu.SemaphoreType.DMA,
    ],
)
def _msg_sc(x_hbm, src_hbm, dst_hbm, ea_hbm, out_hbm, srcr, dstr,
            xrA, xrB, msA, msB, scA, scB, agg_sh, sem_idx, sem_ia, sem_ib,
            sem_sa, sem_sb):
    c = lax.axis_index("c")
    s = lax.axis_index("s")
    w = s * NC + c
    ebase = w * EPT

    def _zero_row(r, t):
        for g in range(D // 16):
            scA[r, pl.ds(g * 16, 16)] = jnp.zeros((16,), jnp.float32)
        return t
    lax.fori_loop(0, CH, _zero_row, 0)
    r0 = s * WB
    for k in range(WB // CH):
        pltpu.sync_copy(scA, agg_sh.at[pl.ds(r0 + k * CH, CH)])
    _tl = WB - (WB // CH) * CH
    pltpu.sync_copy(scA.at[pl.ds(0, _tl)],
                    agg_sh.at[pl.ds(r0 + (WB // CH) * CH, _tl)])

    @pl.when(s == NS - 1)
    def _zero_tail():
        pltpu.sync_copy(scA.at[pl.ds(0, REM)],
                        agg_sh.at[pl.ds(NS * WB, REM)])
    plsc.subcore_barrier()

    def issue_idx(j, q):
        off = ebase + j * CH
        pltpu.async_copy(src_hbm.at[pl.ds(off, CH)], srcr.at[q], sem_idx)
        pltpu.async_copy(dst_hbm.at[pl.ds(off, CH)], dstr.at[q], sem_idx)

    def drain_idx(q):
        pltpu.make_async_copy(src_hbm.at[pl.ds(0, CH)], srcr.at[q], sem_idx).wait()
        pltpu.make_async_copy(src_hbm.at[pl.ds(0, CH)], dstr.at[q], sem_idx).wait()

    def issue_in(j, xr, ms, q, sem):
        pltpu.async_copy(x_hbm.at[srcr.at[q]], xr, sem)
        pltpu.async_copy(ea_hbm.at[w * NCHUNK + j], ms, sem)

    def drain_in(xr, ms, sem):
        pltpu.make_async_copy(out_hbm.at[0].at[pl.ds(0, CH)], xr, sem).wait()
        pltpu.make_async_copy(ea_hbm.at[0], ms, sem).wait()

    def drain_sc(sem):
        pltpu.make_async_copy(out_hbm.at[0].at[pl.ds(0, CH)], scA, sem).wait()

    _HI = jnp.int32(-65536)
    _bc = lambda v: jax.lax.bitcast_convert_type(v, jnp.float32)

    def compute(xr, ms, sc):
        @plsc.parallel_loop(0, CH // 2, step=1, unroll=2)
        def _row(r2):
            for half in range(2):
                for k in range(D // 32):
                    ve = ms[r2, pl.ds(half * (D // 2) + k * 16, 16)]
                    ca = _bc(lax.shift_left(ve, 16))
                    cb = _bc(jnp.bitwise_and(ve, _HI))
                    e = 2 * r2 + half
                    sc[e, pl.ds(k * 32, 16)] = jnp.maximum(
                        xr[e, pl.ds(k * 32, 16)] + ca, 0.0)
                    sc[e, pl.ds(k * 32 + 16, 16)] = jnp.maximum(
                        xr[e, pl.ds(k * 32 + 16, 16)] + cb, 0.0)

    def scatter(sc, q, sem):
        pltpu.async_copy(sc, agg_sh.at[dstr.at[q]], sem, add=True)

    for j in range(4):
        issue_idx(j, j)
    drain_idx(0)
    issue_in(0, xrA, msA, 0, sem_ia)

    def _pair(t, tok):
        j0 = 2 * t
        j1 = j0 + 1
        q0 = jnp.bitwise_and(j0, 7)
        q1 = jnp.bitwise_and(j1, 7)

        drain_idx(q1)
        issue_in(j1, xrB, msB, q1, sem_ib)

        @pl.when(j0 + 4 < NCHUNK)
        def _():
            issue_idx(j0 + 4, jnp.bitwise_and(j0 + 4, 7))

        @pl.when(j0 + 5 < NCHUNK)
        def _():
            issue_idx(j0 + 5, jnp.bitwise_and(j0 + 5, 7))

        drain_in(xrA, msA, sem_ia)

        @pl.when(t >= 1)
        def _():
            drain_sc(sem_sa)

        compute(xrA, msA, scA)
        scatter(scA, q0, sem_sa)

        @pl.when(j0 + 2 < NCHUNK)
        def _():
            drain_idx(jnp.bitwise_and(j0 + 2, 7))
            issue_in(j0 + 2, xrA, msA, jnp.bitwise_and(j0 + 2, 7), sem_ia)

        drain_in(xrB, msB, sem_ib)

        @pl.when(t >= 1)
        def _():
            drain_sc(sem_sb)

        compute(xrB, msB, scB)
        scatter(scB, q1, sem_sb)
        return tok
    lax.fori_loop(0, NCHUNK // 2, _pair, 0)
    drain_sc(sem_sa)
    drain_sc(sem_sb)

    plsc.subcore_barrier()
    pltpu.sync_copy(agg_sh.at[pl.ds(r0, WB)], out_hbm.at[c].at[pl.ds(r0, WB)])

    @pl.when(s == NS - 1)
    def _write_tail():
        pltpu.sync_copy(agg_sh.at[pl.ds(NS * WB, REM)],
                        out_hbm.at[c].at[pl.ds(NS * WB, REM)])


MLP_B = 1000


def _mlp_body(p0, p1, x, w1, b1, w2, b2, o, *, out_relu):
    sagg = p0[...] + p1[...] + x[...]
    h = jnp.maximum(
        jnp.dot(sagg, w1[...], preferred_element_type=jnp.float32) + b1[...], 0.0)
    y = jnp.dot(h, w2[...], preferred_element_type=jnp.float32) + b2[...]
    if out_relu:
        y = jnp.maximum(y, 0.0)
    o[...] = y + x[...]


def _mlp(partials, x, w1, b1, w2, b2, out_relu):
    row_spec = pl.BlockSpec((MLP_B, D), lambda i: (i, 0))
    full_spec = pl.BlockSpec((D, D), lambda i: (0, 0))
    bias_spec = pl.BlockSpec((1, D), lambda i: (0, 0))
    return pl.pallas_call(
        functools.partial(_mlp_body, out_relu=out_relu),
        grid=(N // MLP_B,),
        in_specs=[row_spec, row_spec, row_spec, full_spec, bias_spec,
                  full_spec, bias_spec],
        out_specs=row_spec,
        out_shape=jax.ShapeDtypeStruct((N, D), jnp.float32),
    )(partials[0], partials[1], x, w1, b1.reshape(1, D), w2, b2.reshape(1, D))


def _pack_bf16(a):
    m = a.shape[0]
    ab = a.reshape(m, D // 32, 2, 16).astype(jnp.bfloat16)
    u = jax.lax.bitcast_convert_type(ab, jnp.uint16).astype(jnp.uint32)
    w = u[:, :, 0, :] | (u[:, :, 1, :] << 16)
    w = jax.lax.bitcast_convert_type(w, jnp.int32)
    return w.reshape(m // CH, CH * (D // 2) // D, D)


def kernel(z, edge_index, edge_attr, emb, W1, b1, W2, b2):
    z = z.astype(jnp.int32)
    z_pad = jnp.concatenate([z, jnp.zeros((NPAD - N,), jnp.int32)])
    x = _embed_sc(emb, z_pad)[:N]

    src = edge_index[0]
    dst = edge_index[1]
    ea_p = _pack_bf16(edge_attr)

    for i in range(3):
        partials = _msg_sc(x, src, dst, ea_p)
        x = _mlp(partials, x, W1[i], b1[i], W2[i], b2[i], out_relu=(i < 2))
    return x


# speedup vs baseline: 2.0744x; 2.0744x over previous
import functools

import jax
import jax.numpy as jnp
from jax import lax
from jax.experimental import pallas as pl
from jax.experimental.pallas import tpu as pltpu
from jax.experimental.pallas import tpu_sc as plsc

N = 10000
E = 320000
D = 128

NC = 2
NS = 16
NW = NC * NS

NPAD = 10240
EMB_PER_W = NPAD // NW
EMB_CH = 80
EMB_NCH = EMB_PER_W // EMB_CH

EPT = E // NW
CH = 40
NCHUNK = EPT // CH
WB = 624
REM = N - NS * WB

_mesh = plsc.VectorSubcoreMesh(core_axis_name="c", subcore_axis_name="s")


@functools.partial(
    pl.kernel,
    out_type=jax.ShapeDtypeStruct((NPAD, D), jnp.float32),
    mesh=_mesh,
    scratch_types=[
        pltpu.VMEM((EMB_CH,), jnp.int32),
        pltpu.VMEM((EMB_CH, D), jnp.float32),
        pltpu.SemaphoreType.DMA,
    ],
)
def _embed_sc(emb_hbm, z_hbm, out_hbm, zi_v, row_v, sem):
    c = lax.axis_index("c")
    s = lax.axis_index("s")
    w = s * NC + c
    base = w * EMB_PER_W
    for k in range(EMB_NCH):
        pltpu.sync_copy(z_hbm.at[pl.ds(base + k * EMB_CH, EMB_CH)], zi_v)
        pltpu.async_copy(emb_hbm.at[zi_v], row_v, sem).wait()
        pltpu.sync_copy(row_v, out_hbm.at[pl.ds(base + k * EMB_CH, EMB_CH)])


@functools.partial(
    pl.kernel,
    out_type=jax.ShapeDtypeStruct((NC, N, D), jnp.float32),
    mesh=_mesh,
    scratch_types=[
        pltpu.VMEM((8, CH), jnp.int32),
        pltpu.VMEM((8, CH), jnp.int32),
        pltpu.VMEM((CH, D), jnp.float32),
        pltpu.VMEM((CH, D), jnp.float32),
        pltpu.VMEM((CH, D), jnp.float32),
        pltpu.VMEM((CH, D), jnp.float32),
        pltpu.VMEM((CH, D), jnp.float32),
        pltpu.VMEM((CH, D), jnp.float32),
        pltpu.VMEM_SHARED((N, D), jnp.float32),
        pltpu.SemaphoreType.DMA,
        pltpu.SemaphoreType.DMA,
        pltpu.SemaphoreType.DMA,
        pltpu.SemaphoreType.DMA,
        pltpu.SemaphoreType.DMA,
    ],
)
def _msg_sc(x_hbm, src_hbm, dst_hbm, ea_hbm, out_hbm, srcr, dstr,
            xrA, xrB, msA, msB, scA, scB, agg_sh, sem_idx, sem_ia, sem_ib,
            sem_sa, sem_sb):
    c = lax.axis_index("c")
    s = lax.axis_index("s")
    w = s * NC + c
    ebase = w * EPT

    def _zero_row(r, t):
        for g in range(D // 16):
            scA[r, pl.ds(g * 16, 16)] = jnp.zeros((16,), jnp.float32)
        return t
    lax.fori_loop(0, CH, _zero_row, 0)
    r0 = s * WB
    for k in range(WB // CH):
        pltpu.sync_copy(scA, agg_sh.at[pl.ds(r0 + k * CH, CH)])
    _tl = WB - (WB // CH) * CH
    pltpu.sync_copy(scA.at[pl.ds(0, _tl)],
                    agg_sh.at[pl.ds(r0 + (WB // CH) * CH, _tl)])

    @pl.when(s == NS - 1)
    def _zero_tail():
        pltpu.sync_copy(scA.at[pl.ds(0, REM)],
                        agg_sh.at[pl.ds(NS * WB, REM)])
    plsc.subcore_barrier()

    def issue_idx(j, q):
        off = ebase + j * CH
        pltpu.async_copy(src_hbm.at[pl.ds(off, CH)], srcr.at[q], sem_idx)
        pltpu.async_copy(dst_hbm.at[pl.ds(off, CH)], dstr.at[q], sem_idx)

    def drain_idx(q):
        pltpu.make_async_copy(src_hbm.at[pl.ds(0, CH)], srcr.at[q], sem_idx).wait()
        pltpu.make_async_copy(src_hbm.at[pl.ds(0, CH)], dstr.at[q], sem_idx).wait()

    def issue_in(j, xr, ms, q, sem):
        pltpu.async_copy(x_hbm.at[srcr.at[q]], xr, sem)
        pltpu.async_copy(ea_hbm.at[pl.ds(ebase + j * CH, CH)], ms, sem)

    def drain_in(xr, ms, sem):
        pltpu.make_async_copy(ea_hbm.at[pl.ds(0, CH)], xr, sem).wait()
        pltpu.make_async_copy(ea_hbm.at[pl.ds(0, CH)], ms, sem).wait()

    def drain_sc(sem):
        pltpu.make_async_copy(out_hbm.at[0].at[pl.ds(0, CH)], scA, sem).wait()

    def compute(xr, ms, sc):
        @plsc.parallel_loop(0, CH, step=1, unroll=4)
        def _row(r):
            for g in range(D // 16):
                sl = pl.ds(g * 16, 16)
                sc[r, sl] = jnp.maximum(ms[r, sl] + xr[r, sl], 0.0)

    def scatter(sc, q, sem):
        pltpu.async_copy(sc, agg_sh.at[dstr.at[q]], sem, add=True)

    for j in range(4):
        issue_idx(j, j)
    drain_idx(0)
    issue_in(0, xrA, msA, 0, sem_ia)

    def _pair(t, tok):
        j0 = 2 * t
        j1 = j0 + 1
        q0 = jnp.bitwise_and(j0, 7)
        q1 = jnp.bitwise_and(j1, 7)

        drain_idx(q1)
        issue_in(j1, xrB, msB, q1, sem_ib)

        @pl.when(j0 + 4 < NCHUNK)
        def _():
            issue_idx(j0 + 4, jnp.bitwise_and(j0 + 4, 7))

        @pl.when(j0 + 5 < NCHUNK)
        def _():
            issue_idx(j0 + 5, jnp.bitwise_and(j0 + 5, 7))

        drain_in(xrA, msA, sem_ia)

        @pl.when(t >= 1)
        def _():
            drain_sc(sem_sa)

        compute(xrA, msA, scA)
        scatter(scA, q0, sem_sa)

        @pl.when(j0 + 2 < NCHUNK)
        def _():
            drain_idx(jnp.bitwise_and(j0 + 2, 7))
            issue_in(j0 + 2, xrA, msA, jnp.bitwise_and(j0 + 2, 7), sem_ia)

        drain_in(xrB, msB, sem_ib)

        @pl.when(t >= 1)
        def _():
            drain_sc(sem_sb)

        compute(xrB, msB, scB)
        scatter(scB, q1, sem_sb)
        return tok
    lax.fori_loop(0, NCHUNK // 2, _pair, 0)
    drain_sc(sem_sa)
    drain_sc(sem_sb)

    plsc.subcore_barrier()
    pltpu.sync_copy(agg_sh.at[pl.ds(r0, WB)], out_hbm.at[c].at[pl.ds(r0, WB)])

    @pl.when(s == NS - 1)
    def _write_tail():
        pltpu.sync_copy(agg_sh.at[pl.ds(NS * WB, REM)],
                        out_hbm.at[c].at[pl.ds(NS * WB, REM)])


MLP_B = 1000


def _mlp_body(p0, p1, x, w1, b1, w2, b2, o, *, out_relu):
    sagg = p0[...] + p1[...] + x[...]
    h = jnp.maximum(
        jnp.dot(sagg, w1[...], preferred_element_type=jnp.float32) + b1[...], 0.0)
    y = jnp.dot(h, w2[...], preferred_element_type=jnp.float32) + b2[...]
    if out_relu:
        y = jnp.maximum(y, 0.0)
    o[...] = y + x[...]


def _mlp(partials, x, w1, b1, w2, b2, out_relu):
    row_spec = pl.BlockSpec((MLP_B, D), lambda i: (i, 0))
    full_spec = pl.BlockSpec((D, D), lambda i: (0, 0))
    bias_spec = pl.BlockSpec((1, D), lambda i: (0, 0))
    return pl.pallas_call(
        functools.partial(_mlp_body, out_relu=out_relu),
        grid=(N // MLP_B,),
        in_specs=[row_spec, row_spec, row_spec, full_spec, bias_spec,
                  full_spec, bias_spec],
        out_specs=row_spec,
        out_shape=jax.ShapeDtypeStruct((N, D), jnp.float32),
    )(partials[0], partials[1], x, w1, b1.reshape(1, D), w2, b2.reshape(1, D))


def kernel(z, edge_index, edge_attr, emb, W1, b1, W2, b2):
    z = z.astype(jnp.int32)
    z_pad = jnp.concatenate([z, jnp.zeros((NPAD - N,), jnp.int32)])
    x = _embed_sc(emb, z_pad)[:N]

    src = edge_index[0]
    dst = edge_index[1]

    for i in range(3):
        partials = _msg_sc(x, src, dst, edge_attr)
        x = _mlp(partials, x, W1[i], b1[i], W2[i], b2[i], out_relu=(i < 2))
    return x


# speedup vs baseline: 2.0968x; 1.0108x over previous
import functools

import jax
import jax.numpy as jnp
from jax import lax
from jax.experimental import pallas as pl
from jax.experimental.pallas import tpu as pltpu
from jax.experimental.pallas import tpu_sc as plsc

N = 10000
E = 320000
D = 128

NC = 2
NS = 16
NW = NC * NS

NPAD = 10240
EMB_PER_W = NPAD // NW
EMB_CH = 80
EMB_NCH = EMB_PER_W // EMB_CH

EPT = E // NW
CH = 40
NCHUNK = EPT // CH
WB = 624
REM = N - NS * WB

_mesh = plsc.VectorSubcoreMesh(core_axis_name="c", subcore_axis_name="s")


@functools.partial(
    pl.kernel,
    out_type=jax.ShapeDtypeStruct((NPAD, D), jnp.float32),
    mesh=_mesh,
    scratch_types=[
        pltpu.VMEM((EMB_CH,), jnp.int32),
        pltpu.VMEM((EMB_CH,), jnp.int32),
        pltpu.VMEM((EMB_CH, D), jnp.float32),
        pltpu.VMEM((EMB_CH, D), jnp.float32),
        pltpu.SemaphoreType.DMA,
        pltpu.SemaphoreType.DMA,
        pltpu.SemaphoreType.DMA,
    ],
)
def _embed_sc(emb_hbm, z_hbm, out_hbm, zi0, zi1, row0, row1, sem_i, sem_g,
              sem_w):
    c = lax.axis_index("c")
    s = lax.axis_index("s")
    w = s * NC + c
    base = w * EMB_PER_W
    zi = (zi0, zi1)
    row = (row0, row1)
    pltpu.sync_copy(z_hbm.at[pl.ds(base, EMB_CH)], zi0)
    pltpu.async_copy(emb_hbm.at[zi0], row0, sem_g)
    for k in range(EMB_NCH):
        b = k % 2
        if k + 1 < EMB_NCH:
            nb = (k + 1) % 2
            pltpu.sync_copy(
                z_hbm.at[pl.ds(base + (k + 1) * EMB_CH, EMB_CH)], zi[nb])
        pltpu.make_async_copy(emb_hbm.at[pl.ds(0, EMB_CH)], row[b], sem_g).wait()
        if k >= 1:
            pltpu.make_async_copy(emb_hbm.at[pl.ds(0, EMB_CH)], row[1 - b],
                                  sem_w).wait()
        if k + 1 < EMB_NCH:
            pltpu.async_copy(emb_hbm.at[zi[nb]], row[nb], sem_g)
        pltpu.async_copy(row[b], out_hbm.at[pl.ds(base + k * EMB_CH, EMB_CH)],
                         sem_w)
    pltpu.make_async_copy(emb_hbm.at[pl.ds(0, EMB_CH)], row[(EMB_NCH - 1) % 2],
                          sem_w).wait()


@functools.partial(
    pl.kernel,
    out_type=jax.ShapeDtypeStruct((NC, N, D), jnp.float32),
    mesh=_mesh,
    scratch_types=[
        pltpu.VMEM((8, CH), jnp.int32),
        pltpu.VMEM((8, CH), jnp.int32),
        pltpu.VMEM((CH, D), jnp.float32),
        pltpu.VMEM((CH, D), jnp.float32),
        pltpu.VMEM((CH, D), jnp.float32),
        pltpu.VMEM((CH, D), jnp.float32),
        pltpu.VMEM((CH, D), jnp.float32),
        pltpu.VMEM((CH, D), jnp.float32),
        pltpu.VMEM_SHARED((N, D), jnp.float32),
        pltpu.SemaphoreType.DMA,
        pltpu.SemaphoreType.DMA,
        pltpu.SemaphoreType.DMA,
        pltpu.SemaphoreType.DMA,
        pltpu.SemaphoreType.DMA,
    ],
)
def _msg_sc(x_hbm, src_hbm, dst_hbm, ea_hbm, out_hbm, srcr, dstr,
            xrA, xrB, msA, msB, scA, scB, agg_sh, sem_idx, sem_ia, sem_ib,
            sem_sa, sem_sb):
    c = lax.axis_index("c")
    s = lax.axis_index("s")
    w = s * NC + c
    ebase = w * EPT

    @plsc.parallel_loop(0, CH, step=1, unroll=4)
    def _zero_row(r):
        for g in range(D // 16):
            scA[r, pl.ds(g * 16, 16)] = jnp.zeros((16,), jnp.float32)
    r0 = s * WB
    for k in range(WB // CH):
        pltpu.async_copy(scA, agg_sh.at[pl.ds(r0 + k * CH, CH)], sem_sa)
    _tl = WB - (WB // CH) * CH
    pltpu.async_copy(scA.at[pl.ds(0, _tl)],
                     agg_sh.at[pl.ds(r0 + (WB // CH) * CH, _tl)], sem_sa)

    @pl.when(s == NS - 1)
    def _zero_tail():
        pltpu.async_copy(scA.at[pl.ds(0, REM)],
                         agg_sh.at[pl.ds(NS * WB, REM)], sem_sa)
    for k in range(WB // CH):
        pltpu.make_async_copy(ea_hbm.at[pl.ds(0, CH)], scA, sem_sa).wait()
    pltpu.make_async_copy(ea_hbm.at[pl.ds(0, _tl)], scA.at[pl.ds(0, _tl)],
                          sem_sa).wait()

    @pl.when(s == NS - 1)
    def _zero_tail_wait():
        pltpu.make_async_copy(ea_hbm.at[pl.ds(0, REM)], scA.at[pl.ds(0, REM)],
                              sem_sa).wait()
    plsc.subcore_barrier()

    def issue_idx(j, q):
        off = ebase + j * CH
        pltpu.async_copy(src_hbm.at[pl.ds(off, CH)], srcr.at[q], sem_idx)
        pltpu.async_copy(dst_hbm.at[pl.ds(off, CH)], dstr.at[q], sem_idx)

    def drain_idx(q):
        pltpu.make_async_copy(src_hbm.at[pl.ds(0, CH)], srcr.at[q], sem_idx).wait()
        pltpu.make_async_copy(src_hbm.at[pl.ds(0, CH)], dstr.at[q], sem_idx).wait()

    def issue_in(j, xr, ms, q, sem):
        pltpu.async_copy(x_hbm.at[srcr.at[q]], xr, sem)
        pltpu.async_copy(ea_hbm.at[pl.ds(ebase + j * CH, CH)], ms, sem)

    def drain_in(xr, ms, sem):
        pltpu.make_async_copy(ea_hbm.at[pl.ds(0, CH)], xr, sem).wait()
        pltpu.make_async_copy(ea_hbm.at[pl.ds(0, CH)], ms, sem).wait()

    def drain_sc(sem):
        pltpu.make_async_copy(out_hbm.at[0].at[pl.ds(0, CH)], scA, sem).wait()

    def compute(xr, ms, sc):
        @plsc.parallel_loop(0, CH, step=1, unroll=8)
        def _row(r):
            for g in range(D // 16):
                sl = pl.ds(g * 16, 16)
                sc[r, sl] = jnp.maximum(ms[r, sl] + xr[r, sl], 0.0)

    def scatter(sc, q, sem):
        pltpu.async_copy(sc, agg_sh.at[dstr.at[q]], sem, add=True)

    for j in range(4):
        issue_idx(j, j)
    drain_idx(0)
    issue_in(0, xrA, msA, 0, sem_ia)

    def _pair(t, tok):
        j0 = 2 * t
        j1 = j0 + 1
        q0 = jnp.bitwise_and(j0, 7)
        q1 = jnp.bitwise_and(j1, 7)

        drain_idx(q1)
        issue_in(j1, xrB, msB, q1, sem_ib)

        @pl.when(j0 + 4 < NCHUNK)
        def _():
            issue_idx(j0 + 4, jnp.bitwise_and(j0 + 4, 7))

        @pl.when(j0 + 5 < NCHUNK)
        def _():
            issue_idx(j0 + 5, jnp.bitwise_and(j0 + 5, 7))

        drain_in(xrA, msA, sem_ia)

        @pl.when(t >= 1)
        def _():
            drain_sc(sem_sa)

        compute(xrA, msA, scA)
        scatter(scA, q0, sem_sa)

        @pl.when(j0 + 2 < NCHUNK)
        def _():
            drain_idx(jnp.bitwise_and(j0 + 2, 7))
            issue_in(j0 + 2, xrA, msA, jnp.bitwise_and(j0 + 2, 7), sem_ia)

        drain_in(xrB, msB, sem_ib)

        @pl.when(t >= 1)
        def _():
            drain_sc(sem_sb)

        compute(xrB, msB, scB)
        scatter(scB, q1, sem_sb)
        return tok
    lax.fori_loop(0, NCHUNK // 2, _pair, 0)
    drain_sc(sem_sa)
    drain_sc(sem_sb)

    plsc.subcore_barrier()
    pltpu.sync_copy(agg_sh.at[pl.ds(r0, WB)], out_hbm.at[c].at[pl.ds(r0, WB)])

    @pl.when(s == NS - 1)
    def _write_tail():
        pltpu.sync_copy(agg_sh.at[pl.ds(NS * WB, REM)],
                        out_hbm.at[c].at[pl.ds(NS * WB, REM)])


MLP_B = 2000


def _mlp_body(p0, p1, x, w1, b1, w2, b2, o, *, out_relu):
    sagg = p0[...] + p1[...] + x[...]
    h = jnp.maximum(
        jnp.dot(sagg, w1[...], preferred_element_type=jnp.float32) + b1[...], 0.0)
    y = jnp.dot(h, w2[...], preferred_element_type=jnp.float32) + b2[...]
    if out_relu:
        y = jnp.maximum(y, 0.0)
    o[...] = y + x[...]


def _mlp(partials, x, w1, b1, w2, b2, out_relu):
    row_spec = pl.BlockSpec((MLP_B, D), lambda i: (i, 0))
    full_spec = pl.BlockSpec((D, D), lambda i: (0, 0))
    bias_spec = pl.BlockSpec((1, D), lambda i: (0, 0))
    return pl.pallas_call(
        functools.partial(_mlp_body, out_relu=out_relu),
        grid=(N // MLP_B,),
        in_specs=[row_spec, row_spec, row_spec, full_spec, bias_spec,
                  full_spec, bias_spec],
        out_specs=row_spec,
        out_shape=jax.ShapeDtypeStruct((N, D), jnp.float32),
    )(partials[0], partials[1], x, w1, b1.reshape(1, D), w2, b2.reshape(1, D))


def kernel(z, edge_index, edge_attr, emb, W1, b1, W2, b2):
    z = z.astype(jnp.int32)
    z_pad = jnp.concatenate([z, jnp.zeros((NPAD - N,), jnp.int32)])
    x = _embed_sc(emb, z_pad)[:N]

    src = edge_index[0]
    dst = edge_index[1]

    for i in range(3):
        partials = _msg_sc(x, src, dst, edge_attr)
        x = _mlp(partials, x, W1[i], b1[i], W2[i], b2[i], out_relu=(i < 2))
    return x


# speedup vs baseline: 2.1063x; 1.0045x over previous
import functools

import jax
import jax.numpy as jnp
from jax import lax
from jax.experimental import pallas as pl
from jax.experimental.pallas import tpu as pltpu
from jax.experimental.pallas import tpu_sc as plsc

N = 10000
E = 320000
D = 128

NC = 2
NS = 16
NW = NC * NS

NPAD = 10240
EMB_PER_W = NPAD // NW
EMB_CH = 80
EMB_NCH = EMB_PER_W // EMB_CH

EPT = E // NW
CH = 40
NCHUNK = EPT // CH
WB = 624
REM = N - NS * WB

_mesh = plsc.VectorSubcoreMesh(core_axis_name="c", subcore_axis_name="s")


@functools.partial(
    pl.kernel,
    out_type=jax.ShapeDtypeStruct((NPAD, D), jnp.float32),
    mesh=_mesh,
    scratch_types=[
        pltpu.VMEM((EMB_CH,), jnp.int32),
        pltpu.VMEM((EMB_CH,), jnp.int32),
        pltpu.VMEM((EMB_CH, D), jnp.float32),
        pltpu.VMEM((EMB_CH, D), jnp.float32),
        pltpu.SemaphoreType.DMA,
        pltpu.SemaphoreType.DMA,
        pltpu.SemaphoreType.DMA,
    ],
)
def _embed_sc(emb_hbm, z_hbm, out_hbm, zi0, zi1, row0, row1, sem_i, sem_g,
              sem_w):
    c = lax.axis_index("c")
    s = lax.axis_index("s")
    w = s * NC + c
    base = w * EMB_PER_W
    zi = (zi0, zi1)
    row = (row0, row1)
    pltpu.sync_copy(z_hbm.at[pl.ds(base, EMB_CH)], zi0)
    pltpu.async_copy(emb_hbm.at[zi0], row0, sem_g)
    for k in range(EMB_NCH):
        b = k % 2
        if k + 1 < EMB_NCH:
            nb = (k + 1) % 2
            pltpu.sync_copy(
                z_hbm.at[pl.ds(base + (k + 1) * EMB_CH, EMB_CH)], zi[nb])
        pltpu.make_async_copy(emb_hbm.at[pl.ds(0, EMB_CH)], row[b], sem_g).wait()
        if k >= 1:
            pltpu.make_async_copy(emb_hbm.at[pl.ds(0, EMB_CH)], row[1 - b],
                                  sem_w).wait()
        if k + 1 < EMB_NCH:
            pltpu.async_copy(emb_hbm.at[zi[nb]], row[nb], sem_g)
        pltpu.async_copy(row[b], out_hbm.at[pl.ds(base + k * EMB_CH, EMB_CH)],
                         sem_w)
    pltpu.make_async_copy(emb_hbm.at[pl.ds(0, EMB_CH)], row[(EMB_NCH - 1) % 2],
                          sem_w).wait()


@functools.partial(
    pl.kernel,
    out_type=jax.ShapeDtypeStruct((NC, N, D), jnp.float32),
    mesh=_mesh,
    scratch_types=[
        pltpu.VMEM((8, CH), jnp.int32),
        pltpu.VMEM((8, CH), jnp.int32),
        pltpu.VMEM((CH, D), jnp.float32),
        pltpu.VMEM((CH, D), jnp.float32),
        pltpu.VMEM((CH, D), jnp.float32),
        pltpu.VMEM((CH, D), jnp.float32),
        pltpu.VMEM((CH, D), jnp.float32),
        pltpu.VMEM((CH, D), jnp.float32),
        pltpu.VMEM_SHARED((N, D), jnp.float32),
        pltpu.SemaphoreType.DMA,
        pltpu.SemaphoreType.DMA,
        pltpu.SemaphoreType.DMA,
        pltpu.SemaphoreType.DMA,
        pltpu.SemaphoreType.DMA,
    ],
)
def _msg_sc(x_hbm, src_hbm, dst_hbm, ea_hbm, out_hbm, srcr, dstr,
            xrA, xrB, msA, msB, scA, scB, agg_sh, sem_idx, sem_ia, sem_ib,
            sem_sa, sem_sb):
    c = lax.axis_index("c")
    s = lax.axis_index("s")
    w = s * NC + c
    ebase = w * EPT

    def issue_idx(j, q):
        off = ebase + j * CH
        pltpu.async_copy(src_hbm.at[pl.ds(off, CH)], srcr.at[q], sem_idx)
        pltpu.async_copy(dst_hbm.at[pl.ds(off, CH)], dstr.at[q], sem_idx)

    def drain_idx(q):
        pltpu.make_async_copy(src_hbm.at[pl.ds(0, CH)], srcr.at[q], sem_idx).wait()
        pltpu.make_async_copy(src_hbm.at[pl.ds(0, CH)], dstr.at[q], sem_idx).wait()

    def issue_in(j, xr, ms, q, sem):
        pltpu.async_copy(x_hbm.at[srcr.at[q]], xr, sem)
        pltpu.async_copy(ea_hbm.at[pl.ds(ebase + j * CH, CH)], ms, sem)

    def drain_in(xr, ms, sem):
        pltpu.make_async_copy(ea_hbm.at[pl.ds(0, CH)], xr, sem).wait()
        pltpu.make_async_copy(ea_hbm.at[pl.ds(0, CH)], ms, sem).wait()

    def drain_sc(sem):
        pltpu.make_async_copy(out_hbm.at[0].at[pl.ds(0, CH)], scA, sem).wait()

    def compute(xr, ms, sc):
        @plsc.parallel_loop(0, CH, step=1, unroll=8)
        def _row(r):
            for g in range(D // 16):
                sl = pl.ds(g * 16, 16)
                sc[r, sl] = jnp.maximum(ms[r, sl] + xr[r, sl], 0.0)

    def scatter(sc, q, sem):
        pltpu.async_copy(sc, agg_sh.at[dstr.at[q]], sem, add=True)

    for j in range(4):
        issue_idx(j, j)
    drain_idx(0)
    issue_in(0, xrA, msA, 0, sem_ia)

    @plsc.parallel_loop(0, CH, step=1, unroll=4)
    def _zero_row(r):
        for g in range(D // 16):
            scA[r, pl.ds(g * 16, 16)] = jnp.zeros((16,), jnp.float32)
    r0 = s * WB
    for k in range(WB // CH):
        pltpu.async_copy(scA, agg_sh.at[pl.ds(r0 + k * CH, CH)], sem_sa)
    _tl = WB - (WB // CH) * CH
    pltpu.async_copy(scA.at[pl.ds(0, _tl)],
                     agg_sh.at[pl.ds(r0 + (WB // CH) * CH, _tl)], sem_sa)

    @pl.when(s == NS - 1)
    def _zero_tail():
        pltpu.async_copy(scA.at[pl.ds(0, REM)],
                         agg_sh.at[pl.ds(NS * WB, REM)], sem_sa)
    for k in range(WB // CH):
        pltpu.make_async_copy(ea_hbm.at[pl.ds(0, CH)], scA, sem_sa).wait()
    pltpu.make_async_copy(ea_hbm.at[pl.ds(0, _tl)], scA.at[pl.ds(0, _tl)],
                          sem_sa).wait()

    @pl.when(s == NS - 1)
    def _zero_tail_wait():
        pltpu.make_async_copy(ea_hbm.at[pl.ds(0, REM)], scA.at[pl.ds(0, REM)],
                              sem_sa).wait()
    plsc.subcore_barrier()

    def _pair(t, tok):
        j0 = 2 * t
        j1 = j0 + 1
        q0 = jnp.bitwise_and(j0, 7)
        q1 = jnp.bitwise_and(j1, 7)

        drain_idx(q1)
        issue_in(j1, xrB, msB, q1, sem_ib)

        @pl.when(j0 + 4 < NCHUNK)
        def _():
            issue_idx(j0 + 4, jnp.bitwise_and(j0 + 4, 7))

        @pl.when(j0 + 5 < NCHUNK)
        def _():
            issue_idx(j0 + 5, jnp.bitwise_and(j0 + 5, 7))

        drain_in(xrA, msA, sem_ia)

        @pl.when(t >= 1)
        def _():
            drain_sc(sem_sa)

        compute(xrA, msA, scA)
        scatter(scA, q0, sem_sa)

        @pl.when(j0 + 2 < NCHUNK)
        def _():
            drain_idx(jnp.bitwise_and(j0 + 2, 7))
            issue_in(j0 + 2, xrA, msA, jnp.bitwise_and(j0 + 2, 7), sem_ia)

        drain_in(xrB, msB, sem_ib)

        @pl.when(t >= 1)
        def _():
            drain_sc(sem_sb)

        compute(xrB, msB, scB)
        scatter(scB, q1, sem_sb)
        return tok
    lax.fori_loop(0, NCHUNK // 2, _pair, 0)
    drain_sc(sem_sa)
    drain_sc(sem_sb)

    plsc.subcore_barrier()
    pltpu.sync_copy(agg_sh.at[pl.ds(r0, WB)], out_hbm.at[c].at[pl.ds(r0, WB)])

    @pl.when(s == NS - 1)
    def _write_tail():
        pltpu.sync_copy(agg_sh.at[pl.ds(NS * WB, REM)],
                        out_hbm.at[c].at[pl.ds(NS * WB, REM)])


MLP_B = 2000


def _mlp_body(p0, p1, x, w1, b1, w2, b2, o, *, out_relu):
    sagg = p0[...] + p1[...] + x[...]
    h = jnp.maximum(
        jnp.dot(sagg, w1[...], preferred_element_type=jnp.float32) + b1[...], 0.0)
    y = jnp.dot(h, w2[...], preferred_element_type=jnp.float32) + b2[...]
    if out_relu:
        y = jnp.maximum(y, 0.0)
    o[...] = y + x[...]


def _mlp(partials, x, w1, b1, w2, b2, out_relu):
    row_spec = pl.BlockSpec((MLP_B, D), lambda i: (i, 0))
    full_spec = pl.BlockSpec((D, D), lambda i: (0, 0))
    bias_spec = pl.BlockSpec((1, D), lambda i: (0, 0))
    return pl.pallas_call(
        functools.partial(_mlp_body, out_relu=out_relu),
        grid=(N // MLP_B,),
        in_specs=[row_spec, row_spec, row_spec, full_spec, bias_spec,
                  full_spec, bias_spec],
        out_specs=row_spec,
        out_shape=jax.ShapeDtypeStruct((N, D), jnp.float32),
    )(partials[0], partials[1], x, w1, b1.reshape(1, D), w2, b2.reshape(1, D))


def kernel(z, edge_index, edge_attr, emb, W1, b1, W2, b2):
    z = z.astype(jnp.int32)
    z_pad = jnp.concatenate([z, jnp.zeros((NPAD - N,), jnp.int32)])
    x = _embed_sc(emb, z_pad)[:N]

    src = edge_index[0]
    dst = edge_index[1]

    for i in range(3):
        partials = _msg_sc(x, src, dst, edge_attr)
        x = _mlp(partials, x, W1[i], b1[i], W2[i], b2[i], out_relu=(i < 2))
    return x


# speedup vs baseline: 2.1123x; 1.0028x over previous
import functools

import jax
import jax.numpy as jnp
from jax import lax
from jax.experimental import pallas as pl
from jax.experimental.pallas import tpu as pltpu
from jax.experimental.pallas import tpu_sc as plsc

N = 10000
E = 320000
D = 128

NC = 2
NS = 16
NW = NC * NS

NPAD = 10240
EMB_PER_W = NPAD // NW
EMB_CH = 80
EMB_NCH = EMB_PER_W // EMB_CH

EPT = E // NW
CH = 40
NCHUNK = EPT // CH
WB = 624
REM = N - NS * WB

_mesh = plsc.VectorSubcoreMesh(core_axis_name="c", subcore_axis_name="s")


@functools.partial(
    pl.kernel,
    out_type=jax.ShapeDtypeStruct((NPAD, D), jnp.float32),
    mesh=_mesh,
    scratch_types=[
        pltpu.VMEM((EMB_CH,), jnp.int32),
        pltpu.VMEM((EMB_CH,), jnp.int32),
        pltpu.VMEM((EMB_CH, D), jnp.float32),
        pltpu.VMEM((EMB_CH, D), jnp.float32),
        pltpu.SemaphoreType.DMA,
        pltpu.SemaphoreType.DMA,
    ],
)
def _embed_sc(emb_hbm, z_hbm, out_hbm, zi0, zi1, row0, row1, sem_g, sem_w):
    c = lax.axis_index("c")
    s = lax.axis_index("s")
    w = s * NC + c
    base = w * EMB_PER_W
    zi = (zi0, zi1)
    row = (row0, row1)
    pltpu.sync_copy(z_hbm.at[pl.ds(base, EMB_CH)], zi0)
    pltpu.async_copy(emb_hbm.at[zi0], row0, sem_g)
    for k in range(EMB_NCH):
        b = k % 2
        if k + 1 < EMB_NCH:
            nb = (k + 1) % 2
            pltpu.sync_copy(
                z_hbm.at[pl.ds(base + (k + 1) * EMB_CH, EMB_CH)], zi[nb])
        pltpu.make_async_copy(emb_hbm.at[pl.ds(0, EMB_CH)], row[b], sem_g).wait()
        if k >= 1:
            pltpu.make_async_copy(emb_hbm.at[pl.ds(0, EMB_CH)], row[1 - b],
                                  sem_w).wait()
        if k + 1 < EMB_NCH:
            pltpu.async_copy(emb_hbm.at[zi[nb]], row[nb], sem_g)
        pltpu.async_copy(row[b], out_hbm.at[pl.ds(base + k * EMB_CH, EMB_CH)],
                         sem_w)
    pltpu.make_async_copy(emb_hbm.at[pl.ds(0, EMB_CH)], row[(EMB_NCH - 1) % 2],
                          sem_w).wait()


@functools.partial(
    pl.kernel,
    out_type=jax.ShapeDtypeStruct((NC, N, D), jnp.float32),
    mesh=_mesh,
    scratch_types=[
        pltpu.VMEM((8, CH), jnp.int32),
        pltpu.VMEM((8, CH), jnp.int32),
        pltpu.VMEM((CH, D), jnp.float32),
        pltpu.VMEM((CH, D), jnp.float32),
        pltpu.VMEM((CH, D), jnp.float32),
        pltpu.VMEM((CH, D), jnp.float32),
        pltpu.VMEM((CH, D), jnp.float32),
        pltpu.VMEM((CH, D), jnp.float32),
        pltpu.VMEM_SHARED((N, D), jnp.float32),
        pltpu.SemaphoreType.DMA,
        pltpu.SemaphoreType.DMA,
        pltpu.SemaphoreType.DMA,
        pltpu.SemaphoreType.DMA,
        pltpu.SemaphoreType.DMA,
    ],
)
def _msg_sc(x_hbm, src_hbm, dst_hbm, ea_hbm, out_hbm, srcr, dstr,
            xrA, xrB, msA, msB, scA, scB, agg_sh, sem_idx, sem_ia, sem_ib,
            sem_sa, sem_sb):
    c = lax.axis_index("c")
    s = lax.axis_index("s")
    w = s * NC + c
    ebase = w * EPT

    def issue_idx(j, q):
        off = ebase + j * CH
        pltpu.async_copy(src_hbm.at[pl.ds(off, CH)], srcr.at[q], sem_idx)
        pltpu.async_copy(dst_hbm.at[pl.ds(off, CH)], dstr.at[q], sem_idx)

    def drain_idx(q):
        pltpu.make_async_copy(src_hbm.at[pl.ds(0, CH)], srcr.at[q], sem_idx).wait()
        pltpu.make_async_copy(src_hbm.at[pl.ds(0, CH)], dstr.at[q], sem_idx).wait()

    def issue_in(j, xr, ms, q, sem):
        pltpu.async_copy(x_hbm.at[srcr.at[q]], xr, sem)
        pltpu.async_copy(ea_hbm.at[pl.ds(ebase + j * CH, CH)], ms, sem)

    def drain_in(xr, ms, sem):
        pltpu.make_async_copy(ea_hbm.at[pl.ds(0, CH)], xr, sem).wait()
        pltpu.make_async_copy(ea_hbm.at[pl.ds(0, CH)], ms, sem).wait()

    def drain_sc(sem):
        pltpu.make_async_copy(out_hbm.at[0].at[pl.ds(0, CH)], scA, sem).wait()

    def compute(xr, ms, sc):
        @plsc.parallel_loop(0, CH, step=1, unroll=8)
        def _row(r):
            for g in range(D // 16):
                sl = pl.ds(g * 16, 16)
                sc[r, sl] = jnp.maximum(ms[r, sl] + xr[r, sl], 0.0)

    def scatter(sc, q, sem):
        pltpu.async_copy(sc, agg_sh.at[dstr.at[q]], sem, add=True)

    for j in range(4):
        issue_idx(j, j)
    drain_idx(0)
    issue_in(0, xrA, msA, 0, sem_ia)

    @plsc.parallel_loop(0, CH, step=1, unroll=4)
    def _zero_row(r):
        for g in range(D // 16):
            scA[r, pl.ds(g * 16, 16)] = jnp.zeros((16,), jnp.float32)
    r0 = s * WB
    for k in range(WB // CH):
        pltpu.async_copy(scA, agg_sh.at[pl.ds(r0 + k * CH, CH)], sem_sa)
    _tl = WB - (WB // CH) * CH
    pltpu.async_copy(scA.at[pl.ds(0, _tl)],
                     agg_sh.at[pl.ds(r0 + (WB // CH) * CH, _tl)], sem_sa)

    @pl.when(s == NS - 1)
    def _zero_tail():
        pltpu.async_copy(scA.at[pl.ds(0, REM)],
                         agg_sh.at[pl.ds(NS * WB, REM)], sem_sa)
    for k in range(WB // CH):
        pltpu.make_async_copy(ea_hbm.at[pl.ds(0, CH)], scA, sem_sa).wait()
    pltpu.make_async_copy(ea_hbm.at[pl.ds(0, _tl)], scA.at[pl.ds(0, _tl)],
                          sem_sa).wait()

    @pl.when(s == NS - 1)
    def _zero_tail_wait():
        pltpu.make_async_copy(ea_hbm.at[pl.ds(0, REM)], scA.at[pl.ds(0, REM)],
                              sem_sa).wait()
    plsc.subcore_barrier()

    def _pair(t, tok):
        j0 = 2 * t
        j1 = j0 + 1
        q0 = jnp.bitwise_and(j0, 7)
        q1 = jnp.bitwise_and(j1, 7)

        drain_idx(q1)
        issue_in(j1, xrB, msB, q1, sem_ib)

        @pl.when(j0 + 4 < NCHUNK)
        def _():
            issue_idx(j0 + 4, jnp.bitwise_and(j0 + 4, 7))

        @pl.when(j0 + 5 < NCHUNK)
        def _():
            issue_idx(j0 + 5, jnp.bitwise_and(j0 + 5, 7))

        drain_in(xrA, msA, sem_ia)

        @pl.when(t >= 1)
        def _():
            drain_sc(sem_sa)

        compute(xrA, msA, scA)
        scatter(scA, q0, sem_sa)

        @pl.when(j0 + 2 < NCHUNK)
        def _():
            drain_idx(jnp.bitwise_and(j0 + 2, 7))
            issue_in(j0 + 2, xrA, msA, jnp.bitwise_and(j0 + 2, 7), sem_ia)

        drain_in(xrB, msB, sem_ib)

        @pl.when(t >= 1)
        def _():
            drain_sc(sem_sb)

        compute(xrB, msB, scB)
        scatter(scB, q1, sem_sb)
        return tok
    lax.fori_loop(0, NCHUNK // 2, _pair, 0)
    drain_sc(sem_sa)
    drain_sc(sem_sb)

    plsc.subcore_barrier()
    pltpu.sync_copy(agg_sh.at[pl.ds(r0, WB)], out_hbm.at[c].at[pl.ds(r0, WB)])

    @pl.when(s == NS - 1)
    def _write_tail():
        pltpu.sync_copy(agg_sh.at[pl.ds(NS * WB, REM)],
                        out_hbm.at[c].at[pl.ds(NS * WB, REM)])


MLP_B = 2000


def _mlp_body(p0, p1, x, w1, b1, w2, b2, o, *, out_relu):
    sagg = p0[...] + p1[...] + x[...]
    h = jnp.maximum(
        jnp.dot(sagg, w1[...], preferred_element_type=jnp.float32) + b1[...], 0.0)
    y = jnp.dot(h, w2[...], preferred_element_type=jnp.float32) + b2[...]
    if out_relu:
        y = jnp.maximum(y, 0.0)
    o[...] = y + x[...]


def _mlp(partials, x, w1, b1, w2, b2, out_relu):
    row_spec = pl.BlockSpec((MLP_B, D), lambda i: (i, 0))
    full_spec = pl.BlockSpec((D, D), lambda i: (0, 0))
    bias_spec = pl.BlockSpec((1, D), lambda i: (0, 0))
    return pl.pallas_call(
        functools.partial(_mlp_body, out_relu=out_relu),
        grid=(N // MLP_B,),
        in_specs=[row_spec, row_spec, row_spec, full_spec, bias_spec,
                  full_spec, bias_spec],
        out_specs=row_spec,
        out_shape=jax.ShapeDtypeStruct((N, D), jnp.float32),
    )(partials[0], partials[1], x, w1, b1.reshape(1, D), w2, b2.reshape(1, D))


def kernel(z, edge_index, edge_attr, emb, W1, b1, W2, b2):
    z = z.astype(jnp.int32)
    z_pad = jnp.concatenate([z, jnp.zeros((NPAD - N,), jnp.int32)])
    x = _embed_sc(emb, z_pad)[:N]

    src = edge_index[0]
    dst = edge_index[1]

    for i in range(3):
        partials = _msg_sc(x, src, dst, edge_attr)
        x = _mlp(partials, x, W1[i], b1[i], W2[i], b2[i], out_relu=(i < 2))
    return x
